# Initial kernel scaffold; baseline (speedup 1.0000x reference)
import jax
import jax.numpy as jnp
from jax.experimental import pallas as pl


def kernel(history_flow, Wq, Wk):
    raise NotImplementedError("write your pallas kernel here")


# speedup vs baseline: 8.2906x; 8.2906x over previous
from math import sqrt

import jax
import jax.numpy as jnp
from jax.experimental import pallas as pl

_B, _T, _N = 8, 96, 2000
_H = 32
_TOPK = 20
_TAU = 0.5
_ALPHA = 0.15
_NP = 2048
_R = 256

_PREC = jax.lax.Precision.HIGHEST


def _qk_body(ns_ref, wq_ref, wk_ref, q_ref, k_ref):
    ns = ns_ref[0]
    dn = (((1,), (1,)), ((), ()))
    q = jax.lax.dot_general(ns, wq_ref[...], dn,
                            preferred_element_type=jnp.float32, precision=_PREC)
    k = jax.lax.dot_general(ns, wk_ref[...], dn,
                            preferred_element_type=jnp.float32, precision=_PREC)
    qn = jnp.sqrt(jnp.sum(q * q, axis=-1, keepdims=True))
    kn = jnp.sqrt(jnp.sum(k * k, axis=-1, keepdims=True))
    q_ref[0] = q / jnp.maximum(qn, 1e-12)
    k_ref[0] = k / jnp.maximum(kn, 1e-12)


def _adj_body(q_ref, k_ref, x_ref, xt_ref, o_ref):
    q = q_ref[0]
    k = k_ref[0]
    x = x_ref[0]
    dn = (((1,), (1,)), ((), ()))
    logits = jax.lax.dot_general(q, k, dn, preferred_element_type=jnp.float32,
                                 precision=_PREC) * (1.0 / sqrt(_H))
    col = jax.lax.broadcasted_iota(jnp.int32, (_R, _NP), 1)
    neg = jnp.float32(-jnp.inf)
    logits = jnp.where(col < _N, logits, neg)
    m1 = jnp.max(logits, axis=-1, keepdims=True)
    m = m1
    work = logits
    for _ in range(_TOPK - 1):
        work = jnp.where(work >= m, neg, work)
        m = jnp.max(work, axis=-1, keepdims=True)
    kth = m
    p = jnp.where(logits >= kth, jnp.exp((logits - m1) * (1.0 / _TAU)), 0.0)
    s = jnp.sum(p, axis=-1, keepdims=True)
    a = p / s
    prop = jax.lax.dot_general(x, a, dn, preferred_element_type=jnp.float32,
                               precision=_PREC)
    o_ref[0] = xt_ref[0] + _ALPHA * prop


def kernel(history_flow, Wq, Wk):
    x = jnp.pad(history_flow, ((0, 0), (0, 0), (0, _NP - _N)))
    ns = jnp.transpose(x, (0, 2, 1))

    q, k = pl.pallas_call(
        _qk_body,
        grid=(_B,),
        in_specs=[
            pl.BlockSpec((1, _NP, _T), lambda b: (b, 0, 0)),
            pl.BlockSpec((_H, _T), lambda b: (0, 0)),
            pl.BlockSpec((_H, _T), lambda b: (0, 0)),
        ],
        out_specs=[
            pl.BlockSpec((1, _NP, _H), lambda b: (b, 0, 0)),
            pl.BlockSpec((1, _NP, _H), lambda b: (b, 0, 0)),
        ],
        out_shape=[
            jax.ShapeDtypeStruct((_B, _NP, _H), jnp.float32),
            jax.ShapeDtypeStruct((_B, _NP, _H), jnp.float32),
        ],
    )(ns, Wq, Wk)

    out = pl.pallas_call(
        _adj_body,
        grid=(_B, _NP // _R),
        in_specs=[
            pl.BlockSpec((1, _R, _H), lambda b, i: (b, i, 0)),
            pl.BlockSpec((1, _NP, _H), lambda b, i: (b, 0, 0)),
            pl.BlockSpec((1, _T, _NP), lambda b, i: (b, 0, 0)),
            pl.BlockSpec((1, _T, _R), lambda b, i: (b, 0, i)),
        ],
        out_specs=pl.BlockSpec((1, _T, _R), lambda b, i: (b, 0, i)),
        out_shape=jax.ShapeDtypeStruct((_B, _T, _NP), jnp.float32),
    )(q, k, x, x)

    return out[:, :, :_N]


# speedup vs baseline: 16.0897x; 1.9407x over previous
from math import sqrt

import jax
import jax.numpy as jnp
from jax.experimental import pallas as pl

_B, _T, _N = 8, 96, 2000
_H = 32
_TOPK = 20
_TAU = 0.5
_ALPHA = 0.15
_NP = 2048
_R = 256

_PREC = jax.lax.Precision.DEFAULT


def _qk_body(ns_ref, wq_ref, wk_ref, q_ref, k_ref):
    ns = ns_ref[0]
    dn = (((1,), (1,)), ((), ()))
    q = jax.lax.dot_general(ns, wq_ref[...], dn,
                            preferred_element_type=jnp.float32, precision=_PREC)
    k = jax.lax.dot_general(ns, wk_ref[...], dn,
                            preferred_element_type=jnp.float32, precision=_PREC)
    qn = jnp.sqrt(jnp.sum(q * q, axis=-1, keepdims=True))
    kn = jnp.sqrt(jnp.sum(k * k, axis=-1, keepdims=True))
    q_ref[0] = q / jnp.maximum(qn, 1e-12)
    k_ref[0] = k / jnp.maximum(kn, 1e-12)


def _adj_body(q_ref, k_ref, x_ref, xt_ref, o_ref):
    q = q_ref[0]
    k = k_ref[0]
    x = x_ref[0]
    dn = (((1,), (1,)), ((), ()))
    logits = jax.lax.dot_general(q, k, dn, preferred_element_type=jnp.float32,
                                 precision=_PREC) * (1.0 / sqrt(_H))
    col = jax.lax.broadcasted_iota(jnp.int32, (_R, _NP), 1)
    neg = jnp.float32(-jnp.inf)
    logits = jnp.where(col < _N, logits, neg)
    m1 = jnp.max(logits, axis=-1, keepdims=True)
    m = m1
    for _ in range(_TOPK - 1):
        m = jnp.max(jnp.where(logits < m, logits, neg), axis=-1, keepdims=True)
    kth = m
    p = jnp.where(logits >= kth, jnp.exp((logits - m1) * (1.0 / _TAU)), 0.0)
    s = jnp.sum(p, axis=-1, keepdims=True)
    prop = jax.lax.dot_general(x, p, dn, preferred_element_type=jnp.float32,
                               precision=_PREC)
    o_ref[0] = xt_ref[0] + _ALPHA * (prop * jnp.transpose(1.0 / s))


def kernel(history_flow, Wq, Wk):
    x = jnp.pad(history_flow, ((0, 0), (0, 0), (0, _NP - _N)))
    ns = jnp.transpose(x, (0, 2, 1))

    q, k = pl.pallas_call(
        _qk_body,
        grid=(_B,),
        in_specs=[
            pl.BlockSpec((1, _NP, _T), lambda b: (b, 0, 0)),
            pl.BlockSpec((_H, _T), lambda b: (0, 0)),
            pl.BlockSpec((_H, _T), lambda b: (0, 0)),
        ],
        out_specs=[
            pl.BlockSpec((1, _NP, _H), lambda b: (b, 0, 0)),
            pl.BlockSpec((1, _NP, _H), lambda b: (b, 0, 0)),
        ],
        out_shape=[
            jax.ShapeDtypeStruct((_B, _NP, _H), jnp.float32),
            jax.ShapeDtypeStruct((_B, _NP, _H), jnp.float32),
        ],
    )(ns, Wq, Wk)

    out = pl.pallas_call(
        _adj_body,
        grid=(_B, _NP // _R),
        in_specs=[
            pl.BlockSpec((1, _R, _H), lambda b, i: (b, i, 0)),
            pl.BlockSpec((1, _NP, _H), lambda b, i: (b, 0, 0)),
            pl.BlockSpec((1, _T, _NP), lambda b, i: (b, 0, 0)),
            pl.BlockSpec((1, _T, _R), lambda b, i: (b, 0, i)),
        ],
        out_specs=pl.BlockSpec((1, _T, _R), lambda b, i: (b, 0, i)),
        out_shape=jax.ShapeDtypeStruct((_B, _T, _NP), jnp.float32),
    )(q, k, x, x)

    return out[:, :, :_N]


# speedup vs baseline: 19.0857x; 1.1862x over previous
from math import sqrt

import jax
import jax.numpy as jnp
from jax.experimental import pallas as pl

_B, _T, _N = 8, 96, 2000
_H = 32
_TOPK = 20
_TAU = 0.5
_ALPHA = 0.15
_NP = 2048
_R = 256

_PREC = jax.lax.Precision.DEFAULT


def _qk_body(ns_ref, wq_ref, wk_ref, q_ref, k_ref):
    ns = ns_ref[0]
    dn = (((1,), (1,)), ((), ()))
    q = jax.lax.dot_general(ns, wq_ref[...], dn,
                            preferred_element_type=jnp.float32, precision=_PREC)
    k = jax.lax.dot_general(ns, wk_ref[...], dn,
                            preferred_element_type=jnp.float32, precision=_PREC)
    qn = jnp.sqrt(jnp.sum(q * q, axis=-1, keepdims=True))
    kn = jnp.sqrt(jnp.sum(k * k, axis=-1, keepdims=True))
    q_ref[0] = q / jnp.maximum(qn, 1e-12)
    k_ref[0] = k / jnp.maximum(kn, 1e-12)


def _adj_body(q_ref, k_ref, x_ref, xt_ref, o_ref):
    q = q_ref[0]
    k = k_ref[0]
    x = x_ref[0]
    dn = (((1,), (1,)), ((), ()))
    logits = jax.lax.dot_general(q, k, dn, preferred_element_type=jnp.float32,
                                 precision=_PREC) * (1.0 / sqrt(_H))
    col = jax.lax.broadcasted_iota(jnp.int32, (_R, _NP), 1)
    neg = jnp.float32(-jnp.inf)
    logits = jnp.where(col < _N, logits, neg)
    g = logits
    for w in (1024, 512, 256, 128, 64, 32):
        g = jnp.maximum(g[:, :w], g[:, w:2 * w])
    m1 = jnp.max(g, axis=-1, keepdims=True)
    lo = jnp.min(g, axis=-1, keepdims=True)
    hi = m1
    for _ in range(12):
        mid = 0.5 * (lo + hi)
        cnt = jnp.sum(jnp.where(logits >= mid, 1.0, 0.0), axis=-1,
                      keepdims=True)
        pred = cnt >= float(_TOPK)
        lo = jnp.where(pred, mid, lo)
        hi = jnp.where(pred, hi, mid)
    kth = lo
    p = jnp.where(logits >= kth, jnp.exp((logits - m1) * (1.0 / _TAU)), 0.0)
    s = jnp.sum(p, axis=-1, keepdims=True)
    prop = jax.lax.dot_general(x, p, dn, preferred_element_type=jnp.float32,
                               precision=_PREC)
    o_ref[0] = xt_ref[0] + _ALPHA * (prop * jnp.transpose(1.0 / s))


def kernel(history_flow, Wq, Wk):
    x = jnp.pad(history_flow, ((0, 0), (0, 0), (0, _NP - _N)))
    ns = jnp.transpose(x, (0, 2, 1))

    q, k = pl.pallas_call(
        _qk_body,
        grid=(_B,),
        in_specs=[
            pl.BlockSpec((1, _NP, _T), lambda b: (b, 0, 0)),
            pl.BlockSpec((_H, _T), lambda b: (0, 0)),
            pl.BlockSpec((_H, _T), lambda b: (0, 0)),
        ],
        out_specs=[
            pl.BlockSpec((1, _NP, _H), lambda b: (b, 0, 0)),
            pl.BlockSpec((1, _NP, _H), lambda b: (b, 0, 0)),
        ],
        out_shape=[
            jax.ShapeDtypeStruct((_B, _NP, _H), jnp.float32),
            jax.ShapeDtypeStruct((_B, _NP, _H), jnp.float32),
        ],
    )(ns, Wq, Wk)

    out = pl.pallas_call(
        _adj_body,
        grid=(_B, _NP // _R),
        in_specs=[
            pl.BlockSpec((1, _R, _H), lambda b, i: (b, i, 0)),
            pl.BlockSpec((1, _NP, _H), lambda b, i: (b, 0, 0)),
            pl.BlockSpec((1, _T, _NP), lambda b, i: (b, 0, 0)),
            pl.BlockSpec((1, _T, _R), lambda b, i: (b, 0, i)),
        ],
        out_specs=pl.BlockSpec((1, _T, _R), lambda b, i: (b, 0, i)),
        out_shape=jax.ShapeDtypeStruct((_B, _T, _NP), jnp.float32),
    )(q, k, x, x)

    return out[:, :, :_N]


# speedup vs baseline: 19.9406x; 1.0448x over previous
from math import sqrt

import jax
import jax.numpy as jnp
from jax.experimental import pallas as pl
from jax.experimental.pallas import tpu as pltpu

_B, _T, _N = 8, 96, 2000
_H = 32
_TOPK = 20
_TAU = 0.5
_ALPHA = 0.15
_NP = 2048
_R = 256
_KBIN = 12

_PREC = jax.lax.Precision.DEFAULT
_DN_T = (((1,), (0,)), ((), ()))
_DN_H = (((0,), (0,)), ((), ()))
_DN_J = (((1,), (1,)), ((), ()))


def _proj_norm(w, x):
    p = jax.lax.dot_general(w, x, _DN_T, preferred_element_type=jnp.float32,
                            precision=_PREC)
    n = jnp.sqrt(jnp.sum(p * p, axis=0, keepdims=True))
    return p / jnp.maximum(n, 1e-12)


def _body(x_ref, wq_ref, wk_ref, o_ref, kt_ref):
    i = pl.program_id(1)
    colx = jax.lax.broadcasted_iota(jnp.int32, (_T, _NP), 1)
    x = jnp.where(colx < _N, x_ref[0], 0.0)

    @pl.when(i == 0)
    def _():
        kt_ref[...] = _proj_norm(wk_ref[...], x)

    xq = x_ref[0, :, pl.ds(i * _R, _R)]
    qt = _proj_norm(wq_ref[...], xq)

    logits = jax.lax.dot_general(qt, kt_ref[...], _DN_H,
                                 preferred_element_type=jnp.float32,
                                 precision=_PREC) * (1.0 / sqrt(_H))
    col = jax.lax.broadcasted_iota(jnp.int32, (_R, _NP), 1)
    neg = jnp.float32(-jnp.inf)
    logits = jnp.where(col < _N, logits, neg)

    g = logits
    for w in (1024, 512, 256, 128, 64, 32):
        g = jnp.maximum(g[:, :w], g[:, w:2 * w])
    m1 = jnp.max(g, axis=-1, keepdims=True)
    lo = jnp.min(g, axis=-1, keepdims=True)
    hi = m1
    for _ in range(_KBIN):
        mid = 0.5 * (lo + hi)
        cnt = jnp.sum(jnp.where(logits >= mid, 1.0, 0.0), axis=-1,
                      keepdims=True)
        pred = cnt >= float(_TOPK)
        lo = jnp.where(pred, mid, lo)
        hi = jnp.where(pred, hi, mid)

    p = jnp.where(logits >= lo, jnp.exp((logits - m1) * (1.0 / _TAU)), 0.0)
    s = jnp.sum(p, axis=-1, keepdims=True)
    prop = jax.lax.dot_general(x, p, _DN_J, preferred_element_type=jnp.float32,
                               precision=_PREC)
    o_ref[0] = xq + _ALPHA * (prop * jnp.transpose(1.0 / s))


def kernel(history_flow, Wq, Wk):
    return pl.pallas_call(
        _body,
        grid=(_B, _NP // _R),
        in_specs=[
            pl.BlockSpec((1, _T, _NP), lambda b, i: (b, 0, 0)),
            pl.BlockSpec((_H, _T), lambda b, i: (0, 0)),
            pl.BlockSpec((_H, _T), lambda b, i: (0, 0)),
        ],
        out_specs=pl.BlockSpec((1, _T, _R), lambda b, i: (b, 0, i)),
        out_shape=jax.ShapeDtypeStruct((_B, _T, _N), jnp.float32),
        scratch_shapes=[pltpu.VMEM((_H, _NP), jnp.float32)],
    )(history_flow, Wq, Wk)


# speedup vs baseline: 22.1757x; 1.1121x over previous
from math import sqrt

import jax
import jax.numpy as jnp
from jax.experimental import pallas as pl
from jax.experimental.pallas import tpu as pltpu

_B, _T, _N = 8, 96, 2000
_H = 32
_TOPK = 20
_TAU = 0.5
_ALPHA = 0.15
_NP = 2048
_R = 256
_KBIN = 10

_PREC = jax.lax.Precision.DEFAULT
_DN_T = (((1,), (0,)), ((), ()))
_DN_H = (((0,), (0,)), ((), ()))
_DN_J = (((1,), (1,)), ((), ()))


def _proj_norm(w, x):
    p = jax.lax.dot_general(w, x, _DN_T, preferred_element_type=jnp.float32,
                            precision=_PREC)
    n = jnp.sqrt(jnp.sum(p * p, axis=0, keepdims=True))
    return p / jnp.maximum(n, 1e-12)


def _body(x_ref, wq_ref, wk_ref, o_ref, kt_ref, xm_ref):
    i = pl.program_id(1)

    @pl.when(i == 0)
    def _():
        colx = jax.lax.broadcasted_iota(jnp.int32, (_T, _NP), 1)
        xm = jnp.where(colx < _N, x_ref[0], 0.0)
        xm_ref[...] = xm
        kt_ref[...] = _proj_norm(wk_ref[...], xm)

    x = xm_ref[...]

    xq = x_ref[0, :, pl.ds(i * _R, _R)]
    qt = _proj_norm(wq_ref[...], xq)

    logits = jax.lax.dot_general(qt, kt_ref[...], _DN_H,
                                 preferred_element_type=jnp.float32,
                                 precision=_PREC) * (1.0 / sqrt(_H))
    col = jax.lax.broadcasted_iota(jnp.int32, (_R, _NP), 1)
    neg = jnp.float32(-jnp.inf)
    logits = jnp.where(col < _N, logits, neg)

    g = logits
    for w in (1024, 512, 256, 128, 64, 32):
        g = jnp.maximum(g[:, :w], g[:, w:2 * w])
    m1 = jnp.max(g, axis=-1, keepdims=True)
    lo = jnp.min(g, axis=-1, keepdims=True)
    hi = m1
    for _ in range(_KBIN):
        mid = 0.5 * (lo + hi)
        cnt = jnp.sum(jnp.where(logits >= mid, 1.0, 0.0), axis=-1,
                      keepdims=True)
        pred = cnt >= float(_TOPK)
        lo = jnp.where(pred, mid, lo)
        hi = jnp.where(pred, hi, mid)

    p = jnp.where(logits >= lo, jnp.exp((logits - m1) * (1.0 / _TAU)), 0.0)
    s = jnp.sum(p, axis=-1, keepdims=True)
    prop = jax.lax.dot_general(x, p, _DN_J, preferred_element_type=jnp.float32,
                               precision=_PREC)
    o_ref[0] = xq + _ALPHA * (prop * jnp.transpose(1.0 / s))


def kernel(history_flow, Wq, Wk):
    return pl.pallas_call(
        _body,
        grid=(_B, _NP // _R),
        in_specs=[
            pl.BlockSpec((1, _T, _NP), lambda b, i: (b, 0, 0)),
            pl.BlockSpec((_H, _T), lambda b, i: (0, 0)),
            pl.BlockSpec((_H, _T), lambda b, i: (0, 0)),
        ],
        out_specs=pl.BlockSpec((1, _T, _R), lambda b, i: (b, 0, i)),
        out_shape=jax.ShapeDtypeStruct((_B, _T, _N), jnp.float32),
        scratch_shapes=[pltpu.VMEM((_H, _NP), jnp.float32),
                        pltpu.VMEM((_T, _NP), jnp.float32)],
    )(history_flow, Wq, Wk)


# speedup vs baseline: 25.6193x; 1.1553x over previous
from math import sqrt

import jax
import jax.numpy as jnp
from jax.experimental import pallas as pl
from jax.experimental.pallas import tpu as pltpu

_B, _T, _N = 8, 96, 2000
_H = 32
_TOPK = 20
_TAU = 0.5
_ALPHA = 0.15
_NP = 2048
_R = 512
_KBIN = 10

_PREC = jax.lax.Precision.DEFAULT
_DN_T = (((1,), (0,)), ((), ()))
_DN_H = (((0,), (0,)), ((), ()))
_DN_J = (((1,), (1,)), ((), ()))


def _proj_norm(w, x):
    p = jax.lax.dot_general(w, x, _DN_T, preferred_element_type=jnp.float32,
                            precision=_PREC)
    n = jnp.sqrt(jnp.sum(p * p, axis=0, keepdims=True))
    return p / jnp.maximum(n, 1e-12)


def _body(x_ref, wq_ref, wk_ref, o_ref, kt_ref, xm_ref):
    i = pl.program_id(1)

    @pl.when(i == 0)
    def _():
        colx = jax.lax.broadcasted_iota(jnp.int32, (_T, _NP), 1)
        xm = jnp.where(colx < _N, x_ref[0], 0.0)
        xm_ref[...] = xm
        kt_ref[...] = _proj_norm(wk_ref[...], xm)

    x = xm_ref[...]

    xq = x_ref[0, :, pl.ds(i * _R, _R)]
    qt = _proj_norm(wq_ref[...], xq)

    logits = jax.lax.dot_general(qt, kt_ref[...], _DN_H,
                                 preferred_element_type=jnp.float32,
                                 precision=_PREC) * (1.0 / sqrt(_H))
    col = jax.lax.broadcasted_iota(jnp.int32, (_R, _NP), 1)
    neg = jnp.float32(-jnp.inf)
    logits = jnp.where(col < _N, logits, neg)

    g = logits
    for w in (1024, 512, 256, 128, 64, 32):
        g = jnp.maximum(g[:, :w], g[:, w:2 * w])
    m1 = jnp.max(g, axis=-1, keepdims=True)
    lo = jnp.min(g, axis=-1, keepdims=True)
    hi = m1
    for _ in range(_KBIN):
        mid = 0.5 * (lo + hi)
        cnt = jnp.sum(jnp.where(logits >= mid, 1.0, 0.0), axis=-1,
                      keepdims=True)
        pred = cnt >= float(_TOPK)
        lo = jnp.where(pred, mid, lo)
        hi = jnp.where(pred, hi, mid)

    p = jnp.where(logits >= lo, jnp.exp((logits - m1) * (1.0 / _TAU)), 0.0)
    s = jnp.sum(p, axis=-1, keepdims=True)
    prop = jax.lax.dot_general(x, p, _DN_J, preferred_element_type=jnp.float32,
                               precision=_PREC)
    o_ref[0] = xq + _ALPHA * (prop * jnp.transpose(1.0 / s))


def kernel(history_flow, Wq, Wk):
    return pl.pallas_call(
        _body,
        grid=(_B, _NP // _R),
        in_specs=[
            pl.BlockSpec((1, _T, _NP), lambda b, i: (b, 0, 0)),
            pl.BlockSpec((_H, _T), lambda b, i: (0, 0)),
            pl.BlockSpec((_H, _T), lambda b, i: (0, 0)),
        ],
        out_specs=pl.BlockSpec((1, _T, _R), lambda b, i: (b, 0, i)),
        out_shape=jax.ShapeDtypeStruct((_B, _T, _N), jnp.float32),
        scratch_shapes=[pltpu.VMEM((_H, _NP), jnp.float32),
                        pltpu.VMEM((_T, _NP), jnp.float32)],
    )(history_flow, Wq, Wk)


# speedup vs baseline: 27.5009x; 1.0734x over previous
from math import sqrt

import jax
import jax.numpy as jnp
from jax.experimental import pallas as pl
from jax.experimental.pallas import tpu as pltpu

_B, _T, _N = 8, 96, 2000
_H = 32
_TOPK = 20
_TAU = 0.5
_ALPHA = 0.15
_NP = 2048
_R = 1024
_KBIN = 10

_PREC = jax.lax.Precision.DEFAULT
_DN_T = (((1,), (0,)), ((), ()))
_DN_H = (((0,), (0,)), ((), ()))
_DN_J = (((1,), (1,)), ((), ()))


def _proj_norm(w, x):
    p = jax.lax.dot_general(w, x, _DN_T, preferred_element_type=jnp.float32,
                            precision=_PREC)
    n = jnp.sqrt(jnp.sum(p * p, axis=0, keepdims=True))
    return p / jnp.maximum(n, 1e-12)


def _body(x_ref, wq_ref, wk_ref, o_ref, kt_ref, xm_ref):
    i = pl.program_id(1)

    @pl.when(i == 0)
    def _():
        colx = jax.lax.broadcasted_iota(jnp.int32, (_T, _NP), 1)
        xm = jnp.where(colx < _N, x_ref[0], 0.0)
        xm_ref[...] = xm
        kt_ref[...] = _proj_norm(wk_ref[...], xm)

    x = xm_ref[...]

    xq = x_ref[0, :, pl.ds(i * _R, _R)]
    qt = _proj_norm(wq_ref[...], xq)

    logits = jax.lax.dot_general(qt, kt_ref[...], _DN_H,
                                 preferred_element_type=jnp.float32,
                                 precision=_PREC) * (1.0 / sqrt(_H))
    col = jax.lax.broadcasted_iota(jnp.int32, (_R, _NP), 1)
    neg = jnp.float32(-jnp.inf)
    logits = jnp.where(col < _N, logits, neg)

    g = logits
    for w in (1024, 512, 256, 128, 64, 32):
        g = jnp.maximum(g[:, :w], g[:, w:2 * w])
    m1 = jnp.max(g, axis=-1, keepdims=True)
    lo = jnp.min(g, axis=-1, keepdims=True)
    hi = m1
    for _ in range(_KBIN):
        mid = 0.5 * (lo + hi)
        cnt = jnp.sum(jnp.where(logits >= mid, 1.0, 0.0), axis=-1,
                      keepdims=True)
        pred = cnt >= float(_TOPK)
        lo = jnp.where(pred, mid, lo)
        hi = jnp.where(pred, hi, mid)

    p = jnp.where(logits >= lo, jnp.exp((logits - m1) * (1.0 / _TAU)), 0.0)
    s = jnp.sum(p, axis=-1, keepdims=True)
    prop = jax.lax.dot_general(x, p, _DN_J, preferred_element_type=jnp.float32,
                               precision=_PREC)
    o_ref[0] = xq + _ALPHA * (prop * jnp.transpose(1.0 / s))


def kernel(history_flow, Wq, Wk):
    return pl.pallas_call(
        _body,
        grid=(_B, _NP // _R),
        in_specs=[
            pl.BlockSpec((1, _T, _NP), lambda b, i: (b, 0, 0)),
            pl.BlockSpec((_H, _T), lambda b, i: (0, 0)),
            pl.BlockSpec((_H, _T), lambda b, i: (0, 0)),
        ],
        out_specs=pl.BlockSpec((1, _T, _R), lambda b, i: (b, 0, i)),
        out_shape=jax.ShapeDtypeStruct((_B, _T, _N), jnp.float32),
        scratch_shapes=[pltpu.VMEM((_H, _NP), jnp.float32),
                        pltpu.VMEM((_T, _NP), jnp.float32)],
    )(history_flow, Wq, Wk)
